# baseline (speedup 1.0000x reference)
import jax
import jax.numpy as jnp
from jax.experimental import pallas as pl


def _memset_body(in_ref, out_ref):
    del in_ref
    out_ref[...] = jnp.zeros_like(out_ref)


def kernel(inputs):
    r = 16000
    flat = pl.pallas_call(
        _memset_body,
        grid=(640000 // r,),
        in_specs=[pl.BlockSpec((8, 20), lambda i: (0, 0))],
        out_specs=pl.BlockSpec((r, 128), lambda i: (i, 0)),
        out_shape=jax.ShapeDtypeStruct((640000, 128), jnp.float32),
    )(inputs.astype(jnp.int32))
    return flat.reshape(4096, 20, 1000)


# speedup vs baseline: 1.0255x; 1.0255x over previous
import jax
import jax.numpy as jnp
from jax import lax
from jax.experimental import pallas as pl
from jax.experimental.pallas import tpu as pltpu
from jax.experimental.pallas import tpu_sc as plsc

DEPTH = 1000
N_ROWS = 4096
N_COLS = 20
TOT = N_ROWS * N_COLS

NUM_CORES = 2
NUM_SUBCORES = 16
NW = NUM_CORES * NUM_SUBCORES
ROWS_PER_W = N_ROWS // NW
QPW = ROWS_PER_W * N_COLS

CROWS = 4
CQ = CROWS * N_COLS
NCHUNK = ROWS_PER_W // CROWS


def _scatter_stateful(refs):
    idx_ref, zc_ref, out_ref = refs
    mesh = plsc.VectorSubcoreMesh(core_axis_name="c", subcore_axis_name="s")

    @pl.core_map(
        mesh,
        compiler_params=pltpu.CompilerParams(
            use_tc_tiling_on_sc=False, needs_layout_passes=False
        ),
    )
    def _():
        def scoped(idx_v, zbuf, sem):
            c = lax.axis_index("c")
            s = lax.axis_index("s")
            wid = s * NUM_CORES + c
            base_q = wid * QPW
            base_n = wid * ROWS_PER_W
            pltpu.sync_copy(idx_ref.at[pl.ds(base_q, QPW)], idx_v)
            pltpu.make_async_copy(zc_ref, zbuf, sem).start()

            ones16 = jnp.full((16,), 1.0, jnp.float32)
            zeros16 = jnp.zeros((16,), jnp.float32)
            avecs, bvecs = [], []
            for t in range(CQ // 16):
                q_rel = t * 16 + lax.iota(jnp.int32, 16)
                avecs.append(q_rel // N_COLS)
                bvecs.append(lax.rem(q_rel, N_COLS))

            pltpu.make_async_copy(zc_ref, zbuf, sem).wait()

            def chunk_body(k, carry):
                for t in range(CQ // 16):
                    d = idx_v[pl.ds(k * CQ + t * 16, 16)]
                    plsc.store_scatter(zbuf, [avecs[t], bvecs[t], d], ones16)
                pltpu.make_async_copy(
                    zbuf, out_ref.at[pl.ds(base_n + k * CROWS, CROWS)], sem
                ).start()
                pltpu.make_async_copy(
                    zbuf, out_ref.at[pl.ds(base_n + k * CROWS, CROWS)], sem
                ).wait()
                for t in range(CQ // 16):
                    d = idx_v[pl.ds(k * CQ + t * 16, 16)]
                    plsc.store_scatter(zbuf, [avecs[t], bvecs[t], d], zeros16)
                return carry

            lax.fori_loop(0, NCHUNK, chunk_body, 0)

        pl.run_scoped(
            scoped,
            pltpu.VMEM((QPW,), jnp.int32),
            pltpu.VMEM((CROWS, N_COLS, DEPTH), jnp.float32),
            pltpu.SemaphoreType.DMA,
        )


def kernel(inputs):
    idx = inputs.astype(jnp.int32).reshape(TOT)
    zchunk = jnp.zeros((CROWS, N_COLS, DEPTH), jnp.float32)
    init = pl.empty((N_ROWS, N_COLS, DEPTH), jnp.float32)
    _, _, out = pl.run_state(_scatter_stateful)((idx, zchunk, init))
    return out
